# baseline (speedup 1.0000x reference)
import functools

import jax
import jax.numpy as jnp
from jax import lax
from jax.experimental import pallas as pl
from jax.experimental.pallas import tpu as pltpu
from jax.experimental.pallas import tpu_sc as plsc

N = 10000
E = 320000
D = 128
D_OUT = 16
NP = 10240

NC = 2
NS = 16
NW = NC * NS
CH = 128
RPT = 80
EPAD = NW * RPT * CH
NBUF = 4
ROWS_T = NP // NS


def _sc_kernel(src2d, dst2d, x_hbm,
               agg_out, w_out, invcnt_out,
               sidx, didx, rows0, ones, invs,
               agg_sp, cnt_sp, w_sp, semg):
    c = lax.axis_index("c")
    s = lax.axis_index("s")
    wid = c * NS + s

    r0 = s * ROWS_T

    def z_body(i, _):
        for k in range(D // 16):
            rows0[i, pl.ds(k * 16, 16)] = jnp.zeros((16,), jnp.float32)
        return 0
    lax.fori_loop(0, CH, z_body, 0)

    def z1_body(i, _):
        invs[pl.ds(i * 16, 16)] = jnp.zeros((16,), jnp.float32)
        return 0
    lax.fori_loop(0, ROWS_T // 16, z1_body, 0)

    for t in range(ROWS_T // CH):
        pltpu.sync_copy(rows0, agg_sp.at[pl.ds(r0 + t * CH, CH)])
    pltpu.sync_copy(invs, cnt_sp.at[pl.ds(r0, ROWS_T)])
    pltpu.sync_copy(invs, w_sp.at[pl.ds(r0, ROWS_T)])
    for i in range(CH // 16):
        ones[pl.ds(i * 16, 16)] = jnp.ones((16,), jnp.float32)

    plsc.subcore_barrier()

    for t in range(NC):
        pltpu.sync_copy(dst2d.at[s * NC + t], didx)

        def cnt_body(j, _):
            pltpu.sync_copy(ones, cnt_sp.at[didx.at[j]], add=True)
            return 0
        lax.fori_loop(0, RPT, cnt_body, 0)

    pltpu.sync_copy(src2d.at[wid], sidx)
    pltpu.sync_copy(dst2d.at[wid], didx)

    def agg_body(j, _):
        pltpu.async_copy(x_hbm.at[sidx.at[j]], rows0, semg).wait()
        pltpu.sync_copy(rows0, agg_sp.at[didx.at[j]], add=True)
        return 0
    lax.fori_loop(0, RPT, agg_body, 0)

    plsc.subcore_barrier()

    pltpu.sync_copy(cnt_sp.at[pl.ds(r0, ROWS_T)], invs)

    def inv_body(i, _):
        v = invs[pl.ds(i * 16, 16)]
        invs[pl.ds(i * 16, 16)] = 1.0 / jnp.maximum(v, 1.0)
        return 0
    lax.fori_loop(0, ROWS_T // 16, inv_body, 0)
    pltpu.sync_copy(invs, cnt_sp.at[pl.ds(r0, ROWS_T)])

    @pl.when(c == 0)
    def _():
        pltpu.sync_copy(invs, invcnt_out.at[pl.ds(r0, ROWS_T)])

    plsc.subcore_barrier()

    def wg_body(j, _):
        pltpu.sync_copy(cnt_sp.at[didx.at[j]], rows0.at[j])
        return 0
    lax.fori_loop(0, RPT, wg_body, 0)

    def ws_body(j, _):
        pltpu.sync_copy(rows0.at[j], w_sp.at[sidx.at[j]], add=True)
        return 0
    lax.fori_loop(0, RPT, ws_body, 0)

    plsc.subcore_barrier()

    pltpu.sync_copy(agg_sp.at[pl.ds(r0, ROWS_T)],
                    agg_out.at[pl.ds(c * NP + r0, ROWS_T)])
    pltpu.sync_copy(w_sp.at[pl.ds(r0, ROWS_T)],
                    w_out.at[pl.ds(c * NP + r0, ROWS_T)])


def _sc_aggregate(x_pad, src2d, dst2d):
    kfn = pl.kernel(
        _sc_kernel,
        mesh=plsc.VectorSubcoreMesh(core_axis_name="c", subcore_axis_name="s"),
        out_type=[
            pltpu.HBM((NC * NP, D), jnp.float32),
            pltpu.HBM((NC * NP,), jnp.float32),
            pltpu.HBM((NP,), jnp.float32),
        ],
        scratch_types=[
            pltpu.VMEM((RPT, CH), jnp.int32),
            pltpu.VMEM((RPT, CH), jnp.int32),
            pltpu.VMEM((CH, D), jnp.float32),
            pltpu.VMEM((CH,), jnp.float32),
            pltpu.VMEM((ROWS_T,), jnp.float32),
            pltpu.VMEM_SHARED((NP, D), jnp.float32),
            pltpu.VMEM_SHARED((NP,), jnp.float32),
            pltpu.VMEM_SHARED((NP,), jnp.float32),
            pltpu.SemaphoreType.DMA,
        ],
    )
    return kfn(src2d, dst2d, x_pad)


ROWS_B = 1024
GRID = NP // ROWS_B


def _tc_kernel(x_ref, a0_ref, a1_ref, inv_ref, w0_ref, w1_ref,
               wl1_ref, bl1_ref, wr1_ref, wl2_ref, bl2_ref, wr2_ref,
               out_ref, sh_acc, s2_acc):
    i = pl.program_id(0)

    @pl.when(i == 0)
    def _():
        sh_acc[...] = jnp.zeros((1, D), jnp.float32)
        s2_acc[...] = jnp.zeros((1, D), jnp.float32)
        out_ref[...] = jnp.zeros((1, D_OUT), jnp.float32)

    mm = functools.partial(lax.dot_general,
                           preferred_element_type=jnp.float32,
                           precision=lax.Precision.HIGHEST)
    eye = (lax.broadcasted_iota(jnp.int32, (D, D), 0) ==
           lax.broadcasted_iota(jnp.int32, (D, D), 1)).astype(jnp.float32)
    sh_l = jnp.zeros((1, D), jnp.float32)
    s2_l = jnp.zeros((1, D), jnp.float32)
    for a in range(ROWS_B // D):
        sl = pl.ds(a * D, D)
        agg = a0_ref[0, sl, :] + a1_ref[0, sl, :]
        inv_row = inv_ref[pl.ds(a, 1), :]
        mean = mm(eye * inv_row, agg, (((1,), (0,)), ((), ())))
        hp = mm(mean, wl1_ref[...], (((1,), (1,)), ((), ())))
        hp += mm(x_ref[sl, :], wr1_ref[...], (((1,), (1,)), ((), ())))
        h = jnp.maximum(hp + bl1_ref[...], 0.0)
        row = i * ROWS_B + a * D + lax.broadcasted_iota(jnp.int32, (D, 1), 0)
        h = jnp.where(row < N, h, 0.0)
        w_row = w0_ref[0, pl.ds(a, 1), :] + w1_ref[0, pl.ds(a, 1), :]
        sh_l += jnp.sum(h, axis=0, keepdims=True)
        s2_l += mm(w_row, h, (((1,), (0,)), ((), ())))
    sh_acc[...] += sh_l
    s2_acc[...] += s2_l

    @pl.when(i == GRID - 1)
    def _():
        invn = 1.0 / float(N)
        pooled = lax.dot_general(s2_acc[...] * invn, wl2_ref[...],
                                 (((1,), (1,)), ((), ())),
                                 preferred_element_type=jnp.float32,
                                 precision=lax.Precision.HIGHEST)
        pooled += lax.dot_general(sh_acc[...] * invn, wr2_ref[...],
                                  (((1,), (1,)), ((), ())),
                                  preferred_element_type=jnp.float32,
                                  precision=lax.Precision.HIGHEST)
        pooled += bl2_ref[...]
        m = jnp.max(pooled, axis=-1, keepdims=True)
        e = jnp.exp(pooled - m)
        out_ref[...] = e / jnp.sum(e, axis=-1, keepdims=True)


def _tc_finish(x_pad, agg_parts, w_parts, invcnt,
               W_l1, b_l1, W_r1, W_l2, b_l2, W_r2):
    inv2d = invcnt.reshape(NP // D, D)
    w2d = w_parts.reshape(NC, NP // D, D)
    a3d = agg_parts.reshape(NC, NP, D)
    full = lambda shape: pl.BlockSpec(shape, lambda i: (0,) * len(shape))
    return pl.pallas_call(
        _tc_kernel,
        grid=(GRID,),
        in_specs=[
            pl.BlockSpec((ROWS_B, D), lambda i: (i, 0)),
            pl.BlockSpec((1, ROWS_B, D), lambda i: (0, i, 0)),
            pl.BlockSpec((1, ROWS_B, D), lambda i: (1, i, 0)),
            pl.BlockSpec((ROWS_B // D, D), lambda i: (i, 0)),
            pl.BlockSpec((1, ROWS_B // D, D), lambda i: (0, i, 0)),
            pl.BlockSpec((1, ROWS_B // D, D), lambda i: (1, i, 0)),
            full((D, D)), full((1, D)), full((D, D)),
            full((D_OUT, D)), full((1, D_OUT)), full((D_OUT, D)),
        ],
        out_specs=pl.BlockSpec((1, D_OUT), lambda i: (0, 0)),
        out_shape=jax.ShapeDtypeStruct((1, D_OUT), jnp.float32),
        scratch_shapes=[pltpu.VMEM((1, D), jnp.float32),
                        pltpu.VMEM((1, D), jnp.float32)],
    )(x_pad, a3d, a3d, inv2d, w2d, w2d,
      W_l1, b_l1.reshape(1, D), W_r1, W_l2, b_l2.reshape(1, D_OUT), W_r2)


def kernel(x, edge_index, W_l1, b_l1, W_r1, W_l2, b_l2, W_r2):
    x_pad = jnp.pad(x, ((0, NP - N), (0, 0)))
    epad = jnp.full((2, EPAD - E), N, jnp.int32)
    ei = jnp.concatenate([edge_index, epad], axis=1)
    src2d = ei[0].reshape(NW, RPT, CH)
    dst2d = ei[1].reshape(NW, RPT, CH)
    agg_parts, w_parts, invcnt = _sc_aggregate(x_pad, src2d, dst2d)
    return _tc_finish(x_pad, agg_parts, w_parts, invcnt,
                      W_l1, b_l1, W_r1, W_l2, b_l2, W_r2)


# speedup vs baseline: 2.0532x; 2.0532x over previous
import functools

import jax
import jax.numpy as jnp
from jax import lax
from jax.experimental import pallas as pl
from jax.experimental.pallas import tpu as pltpu
from jax.experimental.pallas import tpu_sc as plsc

N = 10000
E = 320000
D = 128
D_OUT = 16
NP = 10240

NC = 2
NS = 16
NW = NC * NS
CH = 128
RPT = 80
EPAD = NW * RPT * CH
NBUF = 4
ROWS_T = NP // NS


def _sc_kernel(src2d, dst2d, x_hbm,
               agg_out, w_out, invcnt_out,
               sidx, didx, rows0, ones, invs,
               agg_sp, cnt_sp, w_sp, semg):
    c = lax.axis_index("c")
    s = lax.axis_index("s")
    wid = c * NS + s

    r0 = s * ROWS_T

    def z_body(i, _):
        for k in range(D // 16):
            rows0[i, pl.ds(k * 16, 16)] = jnp.zeros((16,), jnp.float32)
        return 0
    lax.fori_loop(0, CH, z_body, 0)

    def z1_body(i, _):
        invs[pl.ds(i * 16, 16)] = jnp.zeros((16,), jnp.float32)
        return 0
    lax.fori_loop(0, ROWS_T // 16, z1_body, 0)

    for t in range(ROWS_T // CH):
        pltpu.sync_copy(rows0, agg_sp.at[pl.ds(r0 + t * CH, CH)])
    pltpu.sync_copy(invs, cnt_sp.at[pl.ds(r0, ROWS_T)])
    pltpu.sync_copy(invs, w_sp.at[pl.ds(r0, ROWS_T)])
    for i in range(CH // 16):
        ones[pl.ds(i * 16, 16)] = jnp.ones((16,), jnp.float32)

    plsc.subcore_barrier()

    for t in range(NC):
        pltpu.sync_copy(dst2d.at[s * NC + t], didx)

        def cnt_body(j, _):
            pltpu.sync_copy(ones, cnt_sp.at[didx.at[j]], add=True)
            return 0
        lax.fori_loop(0, RPT, cnt_body, 0)

    pltpu.sync_copy(src2d.at[wid], sidx)
    pltpu.sync_copy(dst2d.at[wid], didx)

    def agg_body(j, _):
        pltpu.async_copy(x_hbm.at[sidx.at[j]], rows0, semg).wait()
        pltpu.sync_copy(rows0, agg_sp.at[didx.at[j]], add=True)
        return 0
    lax.fori_loop(0, RPT, agg_body, 0)

    plsc.subcore_barrier()

    pltpu.sync_copy(cnt_sp.at[pl.ds(r0, ROWS_T)], invs)

    def inv_body(i, _):
        v = invs[pl.ds(i * 16, 16)]
        invs[pl.ds(i * 16, 16)] = 1.0 / jnp.maximum(v, 1.0)
        return 0
    lax.fori_loop(0, ROWS_T // 16, inv_body, 0)
    pltpu.sync_copy(invs, cnt_sp.at[pl.ds(r0, ROWS_T)])

    @pl.when(c == 0)
    def _():
        pltpu.sync_copy(invs, invcnt_out.at[pl.ds(r0, ROWS_T)])

    plsc.subcore_barrier()

    def wg_body(j, _):
        pltpu.sync_copy(cnt_sp.at[didx.at[j]], rows0.at[j])
        return 0
    lax.fori_loop(0, RPT, wg_body, 0)

    def ws_body(j, _):
        pltpu.sync_copy(rows0.at[j], w_sp.at[sidx.at[j]], add=True)
        return 0
    lax.fori_loop(0, RPT, ws_body, 0)

    plsc.subcore_barrier()

    pltpu.sync_copy(agg_sp.at[pl.ds(r0, ROWS_T)],
                    agg_out.at[pl.ds(c * NP + r0, ROWS_T)])
    pltpu.sync_copy(w_sp.at[pl.ds(r0, ROWS_T)],
                    w_out.at[pl.ds(c * NP + r0, ROWS_T)])


def _sc_aggregate(x_pad, src2d, dst2d):
    kfn = pl.kernel(
        _sc_kernel,
        mesh=plsc.VectorSubcoreMesh(core_axis_name="c", subcore_axis_name="s"),
        out_type=[
            pltpu.HBM((NC * NP, D), jnp.float32),
            pltpu.HBM((NC * NP,), jnp.float32),
            pltpu.HBM((NP,), jnp.float32),
        ],
        scratch_types=[
            pltpu.VMEM((RPT, CH), jnp.int32),
            pltpu.VMEM((RPT, CH), jnp.int32),
            pltpu.VMEM((CH, D), jnp.float32),
            pltpu.VMEM((CH,), jnp.float32),
            pltpu.VMEM((ROWS_T,), jnp.float32),
            pltpu.VMEM_SHARED((NP, D), jnp.float32),
            pltpu.VMEM_SHARED((NP,), jnp.float32),
            pltpu.VMEM_SHARED((NP,), jnp.float32),
            pltpu.SemaphoreType.DMA,
        ],
    )
    return kfn(src2d, dst2d, x_pad)


ROWS_B = 1024
GRID = NP // ROWS_B


def _tc_kernel(x_ref, a0_ref, a1_ref, inv_ref, w0_ref, w1_ref,
               wl1_ref, bl1_ref, wr1_ref, wl2_ref, bl2_ref, wr2_ref,
               out_ref, sh_acc, s2_acc):
    i = pl.program_id(0)

    @pl.when(i == 0)
    def _():
        sh_acc[...] = jnp.zeros((1, D), jnp.float32)
        s2_acc[...] = jnp.zeros((1, D), jnp.float32)
        out_ref[...] = jnp.zeros((1, D_OUT), jnp.float32)

    mm = functools.partial(lax.dot_general,
                           preferred_element_type=jnp.float32,
                           precision=lax.Precision.HIGHEST)
    eye = (lax.broadcasted_iota(jnp.int32, (D, D), 0) ==
           lax.broadcasted_iota(jnp.int32, (D, D), 1)).astype(jnp.float32)
    sh_l = jnp.zeros((1, D), jnp.float32)
    s2_l = jnp.zeros((1, D), jnp.float32)
    for a in range(ROWS_B // D):
        sl = pl.ds(a * D, D)
        agg = a0_ref[0, sl, :] + a1_ref[0, sl, :]
        inv_row = inv_ref[pl.ds(a, 1), :]
        mean = mm(eye * inv_row, agg, (((1,), (0,)), ((), ())))
        hp = mm(mean, wl1_ref[...], (((1,), (1,)), ((), ())))
        hp += mm(x_ref[sl, :], wr1_ref[...], (((1,), (1,)), ((), ())))
        h = jnp.maximum(hp + bl1_ref[...], 0.0)
        row = i * ROWS_B + a * D + lax.broadcasted_iota(jnp.int32, (D, 1), 0)
        h = jnp.where(row < N, h, 0.0)
        w_row = w0_ref[0, pl.ds(a, 1), :] + w1_ref[0, pl.ds(a, 1), :]
        sh_l += jnp.sum(h, axis=0, keepdims=True)
        s2_l += mm(w_row, h, (((1,), (0,)), ((), ())))
    sh_acc[...] += sh_l
    s2_acc[...] += s2_l

    @pl.when(i == GRID - 1)
    def _():
        invn = 1.0 / float(N)
        pooled = lax.dot_general(s2_acc[...] * invn, wl2_ref[...],
                                 (((1,), (1,)), ((), ())),
                                 preferred_element_type=jnp.float32,
                                 precision=lax.Precision.HIGHEST)
        pooled += lax.dot_general(sh_acc[...] * invn, wr2_ref[...],
                                  (((1,), (1,)), ((), ())),
                                  preferred_element_type=jnp.float32,
                                  precision=lax.Precision.HIGHEST)
        pooled += bl2_ref[...]
        m = jnp.max(pooled, axis=-1, keepdims=True)
        e = jnp.exp(pooled - m)
        out_ref[...] = e / jnp.sum(e, axis=-1, keepdims=True)


def _tc_finish(x_pad, agg_parts, w_parts, invcnt,
               W_l1, b_l1, W_r1, W_l2, b_l2, W_r2):
    inv2d = invcnt.reshape(NP // D, D)
    w2d = w_parts.reshape(NC, NP // D, D)
    a3d = agg_parts.reshape(NC, NP, D)
    full = lambda shape: pl.BlockSpec(shape, lambda i: (0,) * len(shape))
    return pl.pallas_call(
        _tc_kernel,
        grid=(GRID,),
        in_specs=[
            pl.BlockSpec((ROWS_B, D), lambda i: (i, 0)),
            pl.BlockSpec((1, ROWS_B, D), lambda i: (0, i, 0)),
            pl.BlockSpec((1, ROWS_B, D), lambda i: (1, i, 0)),
            pl.BlockSpec((ROWS_B // D, D), lambda i: (i, 0)),
            pl.BlockSpec((1, ROWS_B // D, D), lambda i: (0, i, 0)),
            pl.BlockSpec((1, ROWS_B // D, D), lambda i: (1, i, 0)),
            full((D, D)), full((1, D)), full((D, D)),
            full((D_OUT, D)), full((1, D_OUT)), full((D_OUT, D)),
        ],
        out_specs=pl.BlockSpec((1, D_OUT), lambda i: (0, 0)),
        out_shape=jax.ShapeDtypeStruct((1, D_OUT), jnp.float32),
        scratch_shapes=[pltpu.VMEM((1, D), jnp.float32),
                        pltpu.VMEM((1, D), jnp.float32)],
    )(x_pad, a3d, a3d, inv2d, w2d, w2d,
      W_l1, b_l1.reshape(1, D), W_r1, W_l2, b_l2.reshape(1, D_OUT), W_r2)


def kernel(x, edge_index, W_l1, b_l1, W_r1, W_l2, b_l2, W_r2):
    x_pad = jnp.pad(x, ((0, NP - N), (0, 0)))
    spread = N + (jnp.arange(EPAD - E, dtype=jnp.int32) % (NP - N))
    ei = jnp.concatenate([edge_index, jnp.stack([spread, spread])], axis=1)
    src2d = ei[0].reshape(NW, RPT, CH)
    dst2d = ei[1].reshape(NW, RPT, CH)
    agg_parts, w_parts, invcnt = _sc_aggregate(x_pad, src2d, dst2d)
    return _tc_finish(x_pad, agg_parts, w_parts, invcnt,
                      W_l1, b_l1, W_r1, W_l2, b_l2, W_r2)


# speedup vs baseline: 2.2312x; 1.0867x over previous
import functools

import jax
import jax.numpy as jnp
from jax import lax
from jax.experimental import pallas as pl
from jax.experimental.pallas import tpu as pltpu
from jax.experimental.pallas import tpu_sc as plsc

N = 10000
E = 320000
D = 128
D_OUT = 16
NP = 10240

NC = 2
NS = 16
NW = NC * NS
CH = 128
RPT = 80
EPAD = NW * RPT * CH
NBUF = 4
ROWS_T = NP // NS


def _sc_kernel(src2d, dst2d, x_hbm,
               agg_out, w_out, invcnt_out,
               sidx, didx, rows0, ones, invs,
               agg_sp, cnt_sp, w_sp, semg):
    c = lax.axis_index("c")
    s = lax.axis_index("s")
    wid = c * NS + s

    r0 = s * ROWS_T

    def z_body(i, _):
        for k in range(D // 16):
            rows0[i, pl.ds(k * 16, 16)] = jnp.zeros((16,), jnp.float32)
        return 0
    lax.fori_loop(0, CH, z_body, 0)

    def z1_body(i, _):
        invs[pl.ds(i * 16, 16)] = jnp.zeros((16,), jnp.float32)
        return 0
    lax.fori_loop(0, ROWS_T // 16, z1_body, 0)

    for t in range(ROWS_T // CH):
        pltpu.sync_copy(rows0, agg_sp.at[pl.ds(r0 + t * CH, CH)])
    pltpu.sync_copy(invs, cnt_sp.at[pl.ds(r0, ROWS_T)])
    pltpu.sync_copy(invs, w_sp.at[pl.ds(r0, ROWS_T)])
    for i in range(CH // 16):
        ones[pl.ds(i * 16, 16)] = jnp.ones((16,), jnp.float32)

    plsc.subcore_barrier()

    for t in range(NC):
        pltpu.sync_copy(dst2d.at[s * NC + t], didx)

        def cnt_fire(j, _):
            pltpu.async_copy(ones, cnt_sp.at[didx.at[j]], semg, add=True)
            return 0
        lax.fori_loop(0, RPT, cnt_fire, 0)

        def cnt_drain(j, _):
            pltpu.make_async_copy(ones, cnt_sp.at[didx.at[j]], semg).wait()
            return 0
        lax.fori_loop(0, RPT, cnt_drain, 0)

    pltpu.sync_copy(src2d.at[wid], sidx)
    pltpu.sync_copy(dst2d.at[wid], didx)

    def agg_body(j, _):
        pltpu.async_copy(x_hbm.at[sidx.at[j]], rows0, semg).wait()
        pltpu.sync_copy(rows0, agg_sp.at[didx.at[j]], add=True)
        return 0
    lax.fori_loop(0, RPT, agg_body, 0)

    plsc.subcore_barrier()

    pltpu.sync_copy(cnt_sp.at[pl.ds(r0, ROWS_T)], invs)

    def inv_body(i, _):
        v = invs[pl.ds(i * 16, 16)]
        invs[pl.ds(i * 16, 16)] = 1.0 / jnp.maximum(v, 1.0)
        return 0
    lax.fori_loop(0, ROWS_T // 16, inv_body, 0)
    pltpu.sync_copy(invs, cnt_sp.at[pl.ds(r0, ROWS_T)])

    @pl.when(c == 0)
    def _():
        pltpu.sync_copy(invs, invcnt_out.at[pl.ds(r0, ROWS_T)])

    plsc.subcore_barrier()

    def wg_fire(j, _):
        pltpu.async_copy(cnt_sp.at[didx.at[j]], rows0.at[j], semg)
        return 0
    lax.fori_loop(0, RPT, wg_fire, 0)

    def wg_drain(j, _):
        pltpu.make_async_copy(cnt_sp.at[didx.at[j]], rows0.at[j], semg).wait()
        return 0
    lax.fori_loop(0, RPT, wg_drain, 0)

    def ws_fire(j, _):
        pltpu.async_copy(rows0.at[j], w_sp.at[sidx.at[j]], semg, add=True)
        return 0
    lax.fori_loop(0, RPT, ws_fire, 0)

    def ws_drain(j, _):
        pltpu.make_async_copy(rows0.at[j], w_sp.at[sidx.at[j]], semg).wait()
        return 0
    lax.fori_loop(0, RPT, ws_drain, 0)

    plsc.subcore_barrier()

    pltpu.sync_copy(agg_sp.at[pl.ds(r0, ROWS_T)],
                    agg_out.at[pl.ds(c * NP + r0, ROWS_T)])
    pltpu.sync_copy(w_sp.at[pl.ds(r0, ROWS_T)],
                    w_out.at[pl.ds(c * NP + r0, ROWS_T)])


def _sc_aggregate(x_pad, src2d, dst2d):
    kfn = pl.kernel(
        _sc_kernel,
        mesh=plsc.VectorSubcoreMesh(core_axis_name="c", subcore_axis_name="s"),
        out_type=[
            pltpu.HBM((NC * NP, D), jnp.float32),
            pltpu.HBM((NC * NP,), jnp.float32),
            pltpu.HBM((NP,), jnp.float32),
        ],
        scratch_types=[
            pltpu.VMEM((RPT, CH), jnp.int32),
            pltpu.VMEM((RPT, CH), jnp.int32),
            pltpu.VMEM((CH, D), jnp.float32),
            pltpu.VMEM((CH,), jnp.float32),
            pltpu.VMEM((ROWS_T,), jnp.float32),
            pltpu.VMEM_SHARED((NP, D), jnp.float32),
            pltpu.VMEM_SHARED((NP,), jnp.float32),
            pltpu.VMEM_SHARED((NP,), jnp.float32),
            pltpu.SemaphoreType.DMA,
        ],
    )
    return kfn(src2d, dst2d, x_pad)


ROWS_B = 1024
GRID = NP // ROWS_B


def _tc_kernel(x_ref, a0_ref, a1_ref, inv_ref, w0_ref, w1_ref,
               wl1_ref, bl1_ref, wr1_ref, wl2_ref, bl2_ref, wr2_ref,
               out_ref, sh_acc, s2_acc):
    i = pl.program_id(0)

    @pl.when(i == 0)
    def _():
        sh_acc[...] = jnp.zeros((1, D), jnp.float32)
        s2_acc[...] = jnp.zeros((1, D), jnp.float32)
        out_ref[...] = jnp.zeros((1, D_OUT), jnp.float32)

    mm = functools.partial(lax.dot_general,
                           preferred_element_type=jnp.float32,
                           precision=lax.Precision.HIGHEST)
    eye = (lax.broadcasted_iota(jnp.int32, (D, D), 0) ==
           lax.broadcasted_iota(jnp.int32, (D, D), 1)).astype(jnp.float32)
    sh_l = jnp.zeros((1, D), jnp.float32)
    s2_l = jnp.zeros((1, D), jnp.float32)
    for a in range(ROWS_B // D):
        sl = pl.ds(a * D, D)
        agg = a0_ref[0, sl, :] + a1_ref[0, sl, :]
        inv_row = inv_ref[pl.ds(a, 1), :]
        mean = mm(eye * inv_row, agg, (((1,), (0,)), ((), ())))
        hp = mm(mean, wl1_ref[...], (((1,), (1,)), ((), ())))
        hp += mm(x_ref[sl, :], wr1_ref[...], (((1,), (1,)), ((), ())))
        h = jnp.maximum(hp + bl1_ref[...], 0.0)
        row = i * ROWS_B + a * D + lax.broadcasted_iota(jnp.int32, (D, 1), 0)
        h = jnp.where(row < N, h, 0.0)
        w_row = w0_ref[0, pl.ds(a, 1), :] + w1_ref[0, pl.ds(a, 1), :]
        sh_l += jnp.sum(h, axis=0, keepdims=True)
        s2_l += mm(w_row, h, (((1,), (0,)), ((), ())))
    sh_acc[...] += sh_l
    s2_acc[...] += s2_l

    @pl.when(i == GRID - 1)
    def _():
        invn = 1.0 / float(N)
        pooled = lax.dot_general(s2_acc[...] * invn, wl2_ref[...],
                                 (((1,), (1,)), ((), ())),
                                 preferred_element_type=jnp.float32,
                                 precision=lax.Precision.HIGHEST)
        pooled += lax.dot_general(sh_acc[...] * invn, wr2_ref[...],
                                  (((1,), (1,)), ((), ())),
                                  preferred_element_type=jnp.float32,
                                  precision=lax.Precision.HIGHEST)
        pooled += bl2_ref[...]
        m = jnp.max(pooled, axis=-1, keepdims=True)
        e = jnp.exp(pooled - m)
        out_ref[...] = e / jnp.sum(e, axis=-1, keepdims=True)


def _tc_finish(x_pad, agg_parts, w_parts, invcnt,
               W_l1, b_l1, W_r1, W_l2, b_l2, W_r2):
    inv2d = invcnt.reshape(NP // D, D)
    w2d = w_parts.reshape(NC, NP // D, D)
    a3d = agg_parts.reshape(NC, NP, D)
    full = lambda shape: pl.BlockSpec(shape, lambda i: (0,) * len(shape))
    return pl.pallas_call(
        _tc_kernel,
        grid=(GRID,),
        in_specs=[
            pl.BlockSpec((ROWS_B, D), lambda i: (i, 0)),
            pl.BlockSpec((1, ROWS_B, D), lambda i: (0, i, 0)),
            pl.BlockSpec((1, ROWS_B, D), lambda i: (1, i, 0)),
            pl.BlockSpec((ROWS_B // D, D), lambda i: (i, 0)),
            pl.BlockSpec((1, ROWS_B // D, D), lambda i: (0, i, 0)),
            pl.BlockSpec((1, ROWS_B // D, D), lambda i: (1, i, 0)),
            full((D, D)), full((1, D)), full((D, D)),
            full((D_OUT, D)), full((1, D_OUT)), full((D_OUT, D)),
        ],
        out_specs=pl.BlockSpec((1, D_OUT), lambda i: (0, 0)),
        out_shape=jax.ShapeDtypeStruct((1, D_OUT), jnp.float32),
        scratch_shapes=[pltpu.VMEM((1, D), jnp.float32),
                        pltpu.VMEM((1, D), jnp.float32)],
    )(x_pad, a3d, a3d, inv2d, w2d, w2d,
      W_l1, b_l1.reshape(1, D), W_r1, W_l2, b_l2.reshape(1, D_OUT), W_r2)


def kernel(x, edge_index, W_l1, b_l1, W_r1, W_l2, b_l2, W_r2):
    x_pad = jnp.pad(x, ((0, NP - N), (0, 0)))
    spread = N + (jnp.arange(EPAD - E, dtype=jnp.int32) % (NP - N))
    ei = jnp.concatenate([edge_index, jnp.stack([spread, spread])], axis=1)
    src2d = ei[0].reshape(NW, RPT, CH)
    dst2d = ei[1].reshape(NW, RPT, CH)
    agg_parts, w_parts, invcnt = _sc_aggregate(x_pad, src2d, dst2d)
    return _tc_finish(x_pad, agg_parts, w_parts, invcnt,
                      W_l1, b_l1, W_r1, W_l2, b_l2, W_r2)


# speedup vs baseline: 2.4508x; 1.0984x over previous
import functools

import jax
import jax.numpy as jnp
from jax import lax
from jax.experimental import pallas as pl
from jax.experimental.pallas import tpu as pltpu
from jax.experimental.pallas import tpu_sc as plsc

N = 10000
E = 320000
D = 128
D_OUT = 16
NP = 10240

NC = 2
NS = 16
NW = NC * NS
CH = 128
RPT = 80
EPAD = NW * RPT * CH
NBUF = 4
ROWS_T = NP // NS


IBC = 40
NCH = RPT // IBC


def _sc_kernel(src2d, dst2d, x_hbm,
               agg_out, w_out, invcnt_out,
               sidxc, didxc, rows0, rows1, ones, invs,
               agg_sp, cnt_sp, w_sp,
               semg0, semg1, sems0, sems1, semc):
    c = lax.axis_index("c")
    s = lax.axis_index("s")
    wid = c * NS + s

    r0 = s * ROWS_T

    def z_body(i, _):
        for k in range(D // 16):
            rows0[i, pl.ds(k * 16, 16)] = jnp.zeros((16,), jnp.float32)
        return 0
    lax.fori_loop(0, CH, z_body, 0)

    def z1_body(i, _):
        invs[pl.ds(i * 16, 16)] = jnp.zeros((16,), jnp.float32)
        return 0
    lax.fori_loop(0, ROWS_T // 16, z1_body, 0)

    for t in range(ROWS_T // CH):
        pltpu.sync_copy(rows0, agg_sp.at[pl.ds(r0 + t * CH, CH)])
    pltpu.sync_copy(invs, cnt_sp.at[pl.ds(r0, ROWS_T)])
    pltpu.sync_copy(invs, w_sp.at[pl.ds(r0, ROWS_T)])
    for i in range(CH // 16):
        ones[pl.ds(i * 16, 16)] = jnp.ones((16,), jnp.float32)

    plsc.subcore_barrier()

    def cnt_chunk(q, _):
        plane = s * NC + q // NCH
        off = pl.multiple_of((q % NCH) * IBC, IBC)
        pltpu.sync_copy(dst2d.at[plane, pl.ds(off, IBC)], didxc)

        def fire(r, _):
            pltpu.async_copy(ones, cnt_sp.at[didxc.at[r]], semc, add=True)
            return 0
        lax.fori_loop(0, IBC, fire, 0)

        def drain(r, _):
            pltpu.make_async_copy(ones, cnt_sp.at[didxc.at[r]], semc).wait()
            return 0
        lax.fori_loop(0, IBC, drain, 0)
        return 0
    lax.fori_loop(0, NC * NCH, cnt_chunk, 0)

    def g_start(r, buf, sem):
        pltpu.async_copy(x_hbm.at[sidxc.at[r]], buf, sem)

    def g_wait(r, buf, sem):
        pltpu.make_async_copy(x_hbm.at[sidxc.at[r]], buf, sem).wait()

    def s_start(r, buf, sem):
        pltpu.async_copy(buf, agg_sp.at[didxc.at[r]], sem, add=True)

    def s_wait(r, buf, sem):
        pltpu.make_async_copy(buf, agg_sp.at[didxc.at[r]], sem).wait()

    def agg_chunk(q, _):
        off = pl.multiple_of(q * IBC, IBC)
        pltpu.sync_copy(src2d.at[wid, pl.ds(off, IBC)], sidxc)
        pltpu.sync_copy(dst2d.at[wid, pl.ds(off, IBC)], didxc)
        g_start(0, rows0, semg0)
        g_start(1, rows1, semg1)

        def pair(p, _):
            ra = 2 * p
            rb = 2 * p + 1
            g_wait(ra, rows0, semg0)
            s_start(ra, rows0, sems0)
            g_wait(rb, rows1, semg1)
            s_start(rb, rows1, sems1)
            s_wait(ra, rows0, sems0)

            @pl.when(ra + 2 < IBC)
            def _():
                g_start(ra + 2, rows0, semg0)
            s_wait(rb, rows1, sems1)

            @pl.when(rb + 2 < IBC)
            def _():
                g_start(rb + 2, rows1, semg1)
            return 0
        lax.fori_loop(0, IBC // 2, pair, 0)
        return 0
    lax.fori_loop(0, NCH, agg_chunk, 0)

    plsc.subcore_barrier()

    pltpu.sync_copy(cnt_sp.at[pl.ds(r0, ROWS_T)], invs)

    def inv_body(i, _):
        v = invs[pl.ds(i * 16, 16)]
        invs[pl.ds(i * 16, 16)] = 1.0 / jnp.maximum(v, 1.0)
        return 0
    lax.fori_loop(0, ROWS_T // 16, inv_body, 0)
    pltpu.sync_copy(invs, cnt_sp.at[pl.ds(r0, ROWS_T)])

    @pl.when(c == 0)
    def _():
        pltpu.sync_copy(invs, invcnt_out.at[pl.ds(r0, ROWS_T)])

    plsc.subcore_barrier()

    def w_chunk(q, _):
        off = pl.multiple_of(q * IBC, IBC)
        pltpu.sync_copy(src2d.at[wid, pl.ds(off, IBC)], sidxc)
        pltpu.sync_copy(dst2d.at[wid, pl.ds(off, IBC)], didxc)

        def gfire(r, _):
            pltpu.async_copy(cnt_sp.at[didxc.at[r]], rows0.at[r], semc)
            return 0
        lax.fori_loop(0, IBC, gfire, 0)

        def gdrain(r, _):
            pltpu.make_async_copy(cnt_sp.at[didxc.at[r]], rows0.at[r],
                                  semc).wait()
            return 0
        lax.fori_loop(0, IBC, gdrain, 0)

        def sfire(r, _):
            pltpu.async_copy(rows0.at[r], w_sp.at[sidxc.at[r]], semc,
                             add=True)
            return 0
        lax.fori_loop(0, IBC, sfire, 0)

        def sdrain(r, _):
            pltpu.make_async_copy(rows0.at[r], w_sp.at[sidxc.at[r]],
                                  semc).wait()
            return 0
        lax.fori_loop(0, IBC, sdrain, 0)
        return 0
    lax.fori_loop(0, NCH, w_chunk, 0)

    plsc.subcore_barrier()

    pltpu.sync_copy(agg_sp.at[pl.ds(r0, ROWS_T)],
                    agg_out.at[pl.ds(c * NP + r0, ROWS_T)])
    pltpu.sync_copy(w_sp.at[pl.ds(r0, ROWS_T)],
                    w_out.at[pl.ds(c * NP + r0, ROWS_T)])


def _sc_aggregate(x_pad, src2d, dst2d):
    kfn = pl.kernel(
        _sc_kernel,
        mesh=plsc.VectorSubcoreMesh(core_axis_name="c", subcore_axis_name="s"),
        out_type=[
            pltpu.HBM((NC * NP, D), jnp.float32),
            pltpu.HBM((NC * NP,), jnp.float32),
            pltpu.HBM((NP,), jnp.float32),
        ],
        scratch_types=[
            pltpu.VMEM((IBC, CH), jnp.int32),
            pltpu.VMEM((IBC, CH), jnp.int32),
            pltpu.VMEM((CH, D), jnp.float32),
            pltpu.VMEM((CH, D), jnp.float32),
            pltpu.VMEM((CH,), jnp.float32),
            pltpu.VMEM((ROWS_T,), jnp.float32),
            pltpu.VMEM_SHARED((NP, D), jnp.float32),
            pltpu.VMEM_SHARED((NP,), jnp.float32),
            pltpu.VMEM_SHARED((NP,), jnp.float32),
        ] + [pltpu.SemaphoreType.DMA] * 5,
    )
    return kfn(src2d, dst2d, x_pad)


ROWS_B = 1024
GRID = NP // ROWS_B


def _tc_kernel(x_ref, a0_ref, a1_ref, inv_ref, w0_ref, w1_ref,
               wl1_ref, bl1_ref, wr1_ref, wl2_ref, bl2_ref, wr2_ref,
               out_ref, sh_acc, s2_acc):
    i = pl.program_id(0)

    @pl.when(i == 0)
    def _():
        sh_acc[...] = jnp.zeros((1, D), jnp.float32)
        s2_acc[...] = jnp.zeros((1, D), jnp.float32)
        out_ref[...] = jnp.zeros((1, D_OUT), jnp.float32)

    mm = functools.partial(lax.dot_general,
                           preferred_element_type=jnp.float32,
                           precision=lax.Precision.HIGHEST)
    eye = (lax.broadcasted_iota(jnp.int32, (D, D), 0) ==
           lax.broadcasted_iota(jnp.int32, (D, D), 1)).astype(jnp.float32)
    sh_l = jnp.zeros((1, D), jnp.float32)
    s2_l = jnp.zeros((1, D), jnp.float32)
    for a in range(ROWS_B // D):
        sl = pl.ds(a * D, D)
        agg = a0_ref[0, sl, :] + a1_ref[0, sl, :]
        inv_row = inv_ref[pl.ds(a, 1), :]
        mean = mm(eye * inv_row, agg, (((1,), (0,)), ((), ())))
        hp = mm(mean, wl1_ref[...], (((1,), (1,)), ((), ())))
        hp += mm(x_ref[sl, :], wr1_ref[...], (((1,), (1,)), ((), ())))
        h = jnp.maximum(hp + bl1_ref[...], 0.0)
        row = i * ROWS_B + a * D + lax.broadcasted_iota(jnp.int32, (D, 1), 0)
        h = jnp.where(row < N, h, 0.0)
        w_row = w0_ref[0, pl.ds(a, 1), :] + w1_ref[0, pl.ds(a, 1), :]
        sh_l += jnp.sum(h, axis=0, keepdims=True)
        s2_l += mm(w_row, h, (((1,), (0,)), ((), ())))
    sh_acc[...] += sh_l
    s2_acc[...] += s2_l

    @pl.when(i == GRID - 1)
    def _():
        invn = 1.0 / float(N)
        pooled = lax.dot_general(s2_acc[...] * invn, wl2_ref[...],
                                 (((1,), (1,)), ((), ())),
                                 preferred_element_type=jnp.float32,
                                 precision=lax.Precision.HIGHEST)
        pooled += lax.dot_general(sh_acc[...] * invn, wr2_ref[...],
                                  (((1,), (1,)), ((), ())),
                                  preferred_element_type=jnp.float32,
                                  precision=lax.Precision.HIGHEST)
        pooled += bl2_ref[...]
        m = jnp.max(pooled, axis=-1, keepdims=True)
        e = jnp.exp(pooled - m)
        out_ref[...] = e / jnp.sum(e, axis=-1, keepdims=True)


def _tc_finish(x_pad, agg_parts, w_parts, invcnt,
               W_l1, b_l1, W_r1, W_l2, b_l2, W_r2):
    inv2d = invcnt.reshape(NP // D, D)
    w2d = w_parts.reshape(NC, NP // D, D)
    a3d = agg_parts.reshape(NC, NP, D)
    full = lambda shape: pl.BlockSpec(shape, lambda i: (0,) * len(shape))
    return pl.pallas_call(
        _tc_kernel,
        grid=(GRID,),
        in_specs=[
            pl.BlockSpec((ROWS_B, D), lambda i: (i, 0)),
            pl.BlockSpec((1, ROWS_B, D), lambda i: (0, i, 0)),
            pl.BlockSpec((1, ROWS_B, D), lambda i: (1, i, 0)),
            pl.BlockSpec((ROWS_B // D, D), lambda i: (i, 0)),
            pl.BlockSpec((1, ROWS_B // D, D), lambda i: (0, i, 0)),
            pl.BlockSpec((1, ROWS_B // D, D), lambda i: (1, i, 0)),
            full((D, D)), full((1, D)), full((D, D)),
            full((D_OUT, D)), full((1, D_OUT)), full((D_OUT, D)),
        ],
        out_specs=pl.BlockSpec((1, D_OUT), lambda i: (0, 0)),
        out_shape=jax.ShapeDtypeStruct((1, D_OUT), jnp.float32),
        scratch_shapes=[pltpu.VMEM((1, D), jnp.float32),
                        pltpu.VMEM((1, D), jnp.float32)],
    )(x_pad, a3d, a3d, inv2d, w2d, w2d,
      W_l1, b_l1.reshape(1, D), W_r1, W_l2, b_l2.reshape(1, D_OUT), W_r2)


def kernel(x, edge_index, W_l1, b_l1, W_r1, W_l2, b_l2, W_r2):
    x_pad = jnp.pad(x, ((0, NP - N), (0, 0)))
    spread = N + (jnp.arange(EPAD - E, dtype=jnp.int32) % (NP - N))
    ei = jnp.concatenate([edge_index, jnp.stack([spread, spread])], axis=1)
    src2d = ei[0].reshape(NW, RPT, CH)
    dst2d = ei[1].reshape(NW, RPT, CH)
    agg_parts, w_parts, invcnt = _sc_aggregate(x_pad, src2d, dst2d)
    return _tc_finish(x_pad, agg_parts, w_parts, invcnt,
                      W_l1, b_l1, W_r1, W_l2, b_l2, W_r2)


# speedup vs baseline: 2.4537x; 1.0012x over previous
import functools

import jax
import jax.numpy as jnp
from jax import lax
from jax.experimental import pallas as pl
from jax.experimental.pallas import tpu as pltpu
from jax.experimental.pallas import tpu_sc as plsc

N = 10000
E = 320000
D = 128
D_OUT = 16
NP = 10240

NC = 2
NS = 16
NW = NC * NS
CH = 128
RPT = 80
EPAD = NW * RPT * CH
NBUF = 4
ROWS_T = NP // NS


IBC = 40
NCH = RPT // IBC


def _sc_kernel(src2d, dst2d, x_hbm,
               agg_out, w_out, invcnt_out,
               sidxc, didxc, rows0, rows1, ones, invs,
               agg_sp, cnt_sp, w_sp,
               semg0, semg1, sems0, sems1, semc):
    c = lax.axis_index("c")
    s = lax.axis_index("s")
    wid = c * NS + s

    r0 = s * ROWS_T

    def z_body(i, _):
        for k in range(D // 16):
            rows0[i, pl.ds(k * 16, 16)] = jnp.zeros((16,), jnp.float32)
        return 0
    lax.fori_loop(0, CH, z_body, 0)

    def z1_body(i, _):
        invs[pl.ds(i * 16, 16)] = jnp.zeros((16,), jnp.float32)
        return 0
    lax.fori_loop(0, ROWS_T // 16, z1_body, 0)

    for t in range(ROWS_T // CH):
        pltpu.sync_copy(rows0, agg_sp.at[pl.ds(r0 + t * CH, CH)])
    pltpu.sync_copy(invs, cnt_sp.at[pl.ds(r0, ROWS_T)])
    pltpu.sync_copy(invs, w_sp.at[pl.ds(r0, ROWS_T)])
    for i in range(CH // 16):
        ones[pl.ds(i * 16, 16)] = jnp.ones((16,), jnp.float32)

    plsc.subcore_barrier()

    cbufs = (didxc, sidxc)

    def cnt_drain1(r, _):
        pltpu.make_async_copy(ones, cnt_sp.at[didxc.at[r]], semc).wait()
        return 0

    for q in range(NC * NCH):
        buf = cbufs[q % 2]
        plane = s * NC + q // NCH
        off = (q % NCH) * IBC
        if q >= 2:
            lax.fori_loop(0, IBC, cnt_drain1, 0)
        pltpu.sync_copy(dst2d.at[plane, pl.ds(off, IBC)], buf)

        def cnt_fire(r, _, buf=buf):
            pltpu.async_copy(ones, cnt_sp.at[buf.at[r]], semc, add=True)
            return 0
        lax.fori_loop(0, IBC, cnt_fire, 0)
    lax.fori_loop(0, 2 * IBC, cnt_drain1, 0)

    def g_start(r, buf, sem):
        pltpu.async_copy(x_hbm.at[sidxc.at[r]], buf, sem)

    def g_wait(r, buf, sem):
        pltpu.make_async_copy(x_hbm.at[sidxc.at[r]], buf, sem).wait()

    def s_start(r, buf, sem):
        pltpu.async_copy(buf, agg_sp.at[didxc.at[r]], sem, add=True)

    def s_wait(r, buf, sem):
        pltpu.make_async_copy(buf, agg_sp.at[didxc.at[r]], sem).wait()

    def agg_chunk(q, _):
        off = pl.multiple_of(q * IBC, IBC)
        pltpu.sync_copy(src2d.at[wid, pl.ds(off, IBC)], sidxc)
        pltpu.sync_copy(dst2d.at[wid, pl.ds(off, IBC)], didxc)
        g_start(0, rows0, semg0)
        g_start(1, rows1, semg1)

        def pair(p, _):
            ra = 2 * p
            rb = 2 * p + 1
            g_wait(ra, rows0, semg0)
            s_start(ra, rows0, sems0)
            g_wait(rb, rows1, semg1)
            s_start(rb, rows1, sems1)
            s_wait(ra, rows0, sems0)

            @pl.when(ra + 2 < IBC)
            def _():
                g_start(ra + 2, rows0, semg0)
            s_wait(rb, rows1, sems1)

            @pl.when(rb + 2 < IBC)
            def _():
                g_start(rb + 2, rows1, semg1)
            return 0
        lax.fori_loop(0, IBC // 2, pair, 0)
        return 0
    lax.fori_loop(0, NCH, agg_chunk, 0)

    plsc.subcore_barrier()

    pltpu.sync_copy(cnt_sp.at[pl.ds(r0, ROWS_T)], invs)

    def inv_body(i, _):
        v = invs[pl.ds(i * 16, 16)]
        invs[pl.ds(i * 16, 16)] = 1.0 / jnp.maximum(v, 1.0)
        return 0
    lax.fori_loop(0, ROWS_T // 16, inv_body, 0)
    pltpu.sync_copy(invs, cnt_sp.at[pl.ds(r0, ROWS_T)])

    @pl.when(c == 0)
    def _():
        pltpu.sync_copy(invs, invcnt_out.at[pl.ds(r0, ROWS_T)])

    plsc.subcore_barrier()

    def w_chunk(q, _):
        off = pl.multiple_of(q * IBC, IBC)
        pltpu.sync_copy(src2d.at[wid, pl.ds(off, IBC)], sidxc)
        pltpu.sync_copy(dst2d.at[wid, pl.ds(off, IBC)], didxc)

        def gfire(r, _):
            pltpu.async_copy(cnt_sp.at[didxc.at[r]], rows0.at[r], semc)
            return 0
        lax.fori_loop(0, IBC, gfire, 0)

        def gdrain_sfire(r, _):
            pltpu.make_async_copy(cnt_sp.at[didxc.at[r]], rows0.at[r],
                                  semc).wait()
            pltpu.async_copy(rows0.at[r], w_sp.at[sidxc.at[r]], sems0,
                             add=True)
            return 0
        lax.fori_loop(0, IBC, gdrain_sfire, 0)

        def sdrain(r, _):
            pltpu.make_async_copy(rows0.at[r], w_sp.at[sidxc.at[r]],
                                  sems0).wait()
            return 0
        lax.fori_loop(0, IBC, sdrain, 0)
        return 0
    lax.fori_loop(0, NCH, w_chunk, 0)

    plsc.subcore_barrier()

    pltpu.sync_copy(agg_sp.at[pl.ds(r0, ROWS_T)],
                    agg_out.at[pl.ds(c * NP + r0, ROWS_T)])
    pltpu.sync_copy(w_sp.at[pl.ds(r0, ROWS_T)],
                    w_out.at[pl.ds(c * NP + r0, ROWS_T)])


def _sc_aggregate(x_pad, src2d, dst2d):
    kfn = pl.kernel(
        _sc_kernel,
        mesh=plsc.VectorSubcoreMesh(core_axis_name="c", subcore_axis_name="s"),
        out_type=[
            pltpu.HBM((NC * NP, D), jnp.float32),
            pltpu.HBM((NC * NP,), jnp.float32),
            pltpu.HBM((NP,), jnp.float32),
        ],
        scratch_types=[
            pltpu.VMEM((IBC, CH), jnp.int32),
            pltpu.VMEM((IBC, CH), jnp.int32),
            pltpu.VMEM((CH, D), jnp.float32),
            pltpu.VMEM((CH, D), jnp.float32),
            pltpu.VMEM((CH,), jnp.float32),
            pltpu.VMEM((ROWS_T,), jnp.float32),
            pltpu.VMEM_SHARED((NP, D), jnp.float32),
            pltpu.VMEM_SHARED((NP,), jnp.float32),
            pltpu.VMEM_SHARED((NP,), jnp.float32),
        ] + [pltpu.SemaphoreType.DMA] * 5,
    )
    return kfn(src2d, dst2d, x_pad)


ROWS_B = 1024
GRID = NP // ROWS_B


def _tc_kernel(x_ref, a0_ref, a1_ref, inv_ref, w0_ref, w1_ref,
               wl1_ref, bl1_ref, wr1_ref, wl2_ref, bl2_ref, wr2_ref,
               out_ref, sh_acc, s2_acc):
    i = pl.program_id(0)

    @pl.when(i == 0)
    def _():
        sh_acc[...] = jnp.zeros((1, D), jnp.float32)
        s2_acc[...] = jnp.zeros((1, D), jnp.float32)
        out_ref[...] = jnp.zeros((1, D_OUT), jnp.float32)

    mm = functools.partial(lax.dot_general,
                           preferred_element_type=jnp.float32,
                           precision=lax.Precision.HIGHEST)
    eye = (lax.broadcasted_iota(jnp.int32, (D, D), 0) ==
           lax.broadcasted_iota(jnp.int32, (D, D), 1)).astype(jnp.float32)
    sh_l = jnp.zeros((1, D), jnp.float32)
    s2_l = jnp.zeros((1, D), jnp.float32)
    for a in range(ROWS_B // D):
        sl = pl.ds(a * D, D)
        agg = a0_ref[0, sl, :] + a1_ref[0, sl, :]
        inv_row = inv_ref[pl.ds(a, 1), :]
        mean = mm(eye * inv_row, agg, (((1,), (0,)), ((), ())))
        hp = mm(mean, wl1_ref[...], (((1,), (1,)), ((), ())))
        hp += mm(x_ref[sl, :], wr1_ref[...], (((1,), (1,)), ((), ())))
        h = jnp.maximum(hp + bl1_ref[...], 0.0)
        row = i * ROWS_B + a * D + lax.broadcasted_iota(jnp.int32, (D, 1), 0)
        h = jnp.where(row < N, h, 0.0)
        w_row = w0_ref[0, pl.ds(a, 1), :] + w1_ref[0, pl.ds(a, 1), :]
        sh_l += jnp.sum(h, axis=0, keepdims=True)
        s2_l += mm(w_row, h, (((1,), (0,)), ((), ())))
    sh_acc[...] += sh_l
    s2_acc[...] += s2_l

    @pl.when(i == GRID - 1)
    def _():
        invn = 1.0 / float(N)
        pooled = lax.dot_general(s2_acc[...] * invn, wl2_ref[...],
                                 (((1,), (1,)), ((), ())),
                                 preferred_element_type=jnp.float32,
                                 precision=lax.Precision.HIGHEST)
        pooled += lax.dot_general(sh_acc[...] * invn, wr2_ref[...],
                                  (((1,), (1,)), ((), ())),
                                  preferred_element_type=jnp.float32,
                                  precision=lax.Precision.HIGHEST)
        pooled += bl2_ref[...]
        m = jnp.max(pooled, axis=-1, keepdims=True)
        e = jnp.exp(pooled - m)
        out_ref[...] = e / jnp.sum(e, axis=-1, keepdims=True)


def _tc_finish(x_pad, agg_parts, w_parts, invcnt,
               W_l1, b_l1, W_r1, W_l2, b_l2, W_r2):
    inv2d = invcnt.reshape(NP // D, D)
    w2d = w_parts.reshape(NC, NP // D, D)
    a3d = agg_parts.reshape(NC, NP, D)
    full = lambda shape: pl.BlockSpec(shape, lambda i: (0,) * len(shape))
    return pl.pallas_call(
        _tc_kernel,
        grid=(GRID,),
        in_specs=[
            pl.BlockSpec((ROWS_B, D), lambda i: (i, 0)),
            pl.BlockSpec((1, ROWS_B, D), lambda i: (0, i, 0)),
            pl.BlockSpec((1, ROWS_B, D), lambda i: (1, i, 0)),
            pl.BlockSpec((ROWS_B // D, D), lambda i: (i, 0)),
            pl.BlockSpec((1, ROWS_B // D, D), lambda i: (0, i, 0)),
            pl.BlockSpec((1, ROWS_B // D, D), lambda i: (1, i, 0)),
            full((D, D)), full((1, D)), full((D, D)),
            full((D_OUT, D)), full((1, D_OUT)), full((D_OUT, D)),
        ],
        out_specs=pl.BlockSpec((1, D_OUT), lambda i: (0, 0)),
        out_shape=jax.ShapeDtypeStruct((1, D_OUT), jnp.float32),
        scratch_shapes=[pltpu.VMEM((1, D), jnp.float32),
                        pltpu.VMEM((1, D), jnp.float32)],
    )(x_pad, a3d, a3d, inv2d, w2d, w2d,
      W_l1, b_l1.reshape(1, D), W_r1, W_l2, b_l2.reshape(1, D_OUT), W_r2)


def kernel(x, edge_index, W_l1, b_l1, W_r1, W_l2, b_l2, W_r2):
    x_pad = jnp.pad(x, ((0, NP - N), (0, 0)))
    spread = N + (jnp.arange(EPAD - E, dtype=jnp.int32) % (NP - N))
    ei = jnp.concatenate([edge_index, jnp.stack([spread, spread])], axis=1)
    src2d = ei[0].reshape(NW, RPT, CH)
    dst2d = ei[1].reshape(NW, RPT, CH)
    agg_parts, w_parts, invcnt = _sc_aggregate(x_pad, src2d, dst2d)
    return _tc_finish(x_pad, agg_parts, w_parts, invcnt,
                      W_l1, b_l1, W_r1, W_l2, b_l2, W_r2)


# speedup vs baseline: 2.4590x; 1.0022x over previous
import functools

import jax
import jax.numpy as jnp
from jax import lax
from jax.experimental import pallas as pl
from jax.experimental.pallas import tpu as pltpu
from jax.experimental.pallas import tpu_sc as plsc

N = 10000
E = 320000
D = 128
D_OUT = 16
NP = 10240

NC = 2
NS = 16
NW = NC * NS
CH = 128
RPT = 80
EPAD = NW * RPT * CH
ROWS_T = NP // NS


IBC = 40
NCH = RPT // IBC


def _sc_kernel(src2d, dst2d, x_hbm,
               agg_out, w_out, invcnt_out,
               sidxc, didxc, rows0, rows1, ones, invs,
               agg_sp, cnt_sp, w_sp,
               semg0, semg1, sems0, sems1, semc):
    c = lax.axis_index("c")
    s = lax.axis_index("s")
    wid = c * NS + s

    r0 = s * ROWS_T

    def z_body(i, _):
        for k in range(D // 16):
            rows0[i, pl.ds(k * 16, 16)] = jnp.zeros((16,), jnp.float32)
        return 0
    lax.fori_loop(0, CH, z_body, 0)

    def z1_body(i, _):
        invs[pl.ds(i * 16, 16)] = jnp.zeros((16,), jnp.float32)
        return 0
    lax.fori_loop(0, ROWS_T // 16, z1_body, 0)

    for t in range(ROWS_T // CH):
        pltpu.sync_copy(rows0, agg_sp.at[pl.ds(r0 + t * CH, CH)])
    pltpu.sync_copy(invs, cnt_sp.at[pl.ds(r0, ROWS_T)])
    pltpu.sync_copy(invs, w_sp.at[pl.ds(r0, ROWS_T)])
    for i in range(CH // 16):
        ones[pl.ds(i * 16, 16)] = jnp.ones((16,), jnp.float32)

    plsc.subcore_barrier()

    cbufs = (didxc, sidxc)

    def cnt_drain1(r, _):
        pltpu.make_async_copy(ones, cnt_sp.at[didxc.at[r]], semc).wait()
        return 0

    for q in range(NC * NCH):
        buf = cbufs[q % 2]
        plane = s * NC + q // NCH
        off = (q % NCH) * IBC
        if q >= 2:
            lax.fori_loop(0, IBC, cnt_drain1, 0)
        pltpu.sync_copy(dst2d.at[plane, pl.ds(off, IBC)], buf)

        def cnt_fire(r, _, buf=buf):
            pltpu.async_copy(ones, cnt_sp.at[buf.at[r]], semc, add=True)
            return 0
        lax.fori_loop(0, IBC, cnt_fire, 0)
    lax.fori_loop(0, 2 * IBC, cnt_drain1, 0)

    def g_start(r, buf, sem):
        pltpu.async_copy(x_hbm.at[sidxc.at[r]], buf, sem)

    def g_wait(r, buf, sem):
        pltpu.make_async_copy(x_hbm.at[sidxc.at[r]], buf, sem).wait()

    def s_start(r, buf, sem):
        pltpu.async_copy(buf, agg_sp.at[didxc.at[r]], sem, add=True)

    def s_wait(r, buf, sem):
        pltpu.make_async_copy(buf, agg_sp.at[didxc.at[r]], sem).wait()

    def agg_chunk(q, _):
        off = pl.multiple_of(q * IBC, IBC)
        pltpu.sync_copy(src2d.at[wid, pl.ds(off, IBC)], sidxc)
        pltpu.sync_copy(dst2d.at[wid, pl.ds(off, IBC)], didxc)
        g_start(0, rows0, semg0)
        g_start(1, rows1, semg1)

        def pair(p, _):
            ra = 2 * p
            rb = 2 * p + 1
            g_wait(ra, rows0, semg0)
            s_start(ra, rows0, sems0)
            g_wait(rb, rows1, semg1)
            s_start(rb, rows1, sems1)
            s_wait(ra, rows0, sems0)

            @pl.when(ra + 2 < IBC)
            def _():
                g_start(ra + 2, rows0, semg0)
            s_wait(rb, rows1, sems1)

            @pl.when(rb + 2 < IBC)
            def _():
                g_start(rb + 2, rows1, semg1)
            return 0
        lax.fori_loop(0, IBC // 2, pair, 0)
        return 0
    lax.fori_loop(0, NCH, agg_chunk, 0)

    plsc.subcore_barrier()

    pltpu.sync_copy(cnt_sp.at[pl.ds(r0, ROWS_T)], invs)

    def inv_body(i, _):
        v = invs[pl.ds(i * 16, 16)]
        invs[pl.ds(i * 16, 16)] = 1.0 / jnp.maximum(v, 1.0)
        return 0
    lax.fori_loop(0, ROWS_T // 16, inv_body, 0)
    pltpu.sync_copy(invs, cnt_sp.at[pl.ds(r0, ROWS_T)])

    @pl.when(c == 0)
    def _():
        pltpu.sync_copy(invs, invcnt_out.at[pl.ds(r0, ROWS_T)])

    plsc.subcore_barrier()

    def w_chunk(q, _):
        off = pl.multiple_of(q * IBC, IBC)
        pltpu.sync_copy(src2d.at[wid, pl.ds(off, IBC)], sidxc)
        pltpu.sync_copy(dst2d.at[wid, pl.ds(off, IBC)], didxc)

        def gfire(r, _):
            pltpu.async_copy(cnt_sp.at[didxc.at[r]], rows0.at[r], semc)
            return 0
        lax.fori_loop(0, IBC, gfire, 0)

        def gdrain_sfire(r, _):
            pltpu.make_async_copy(cnt_sp.at[didxc.at[r]], rows0.at[r],
                                  semc).wait()
            pltpu.async_copy(rows0.at[r], w_sp.at[sidxc.at[r]], sems0,
                             add=True)
            return 0
        lax.fori_loop(0, IBC, gdrain_sfire, 0)

        def sdrain(r, _):
            pltpu.make_async_copy(rows0.at[r], w_sp.at[sidxc.at[r]],
                                  sems0).wait()
            return 0
        lax.fori_loop(0, IBC, sdrain, 0)
        return 0
    lax.fori_loop(0, NCH, w_chunk, 0)

    plsc.subcore_barrier()

    pltpu.sync_copy(agg_sp.at[pl.ds(r0, ROWS_T)],
                    agg_out.at[pl.ds(c * NP + r0, ROWS_T)])
    pltpu.sync_copy(w_sp.at[pl.ds(r0, ROWS_T)],
                    w_out.at[pl.ds(c * NP + r0, ROWS_T)])


def _sc_aggregate(x_pad, src2d, dst2d):
    kfn = pl.kernel(
        _sc_kernel,
        mesh=plsc.VectorSubcoreMesh(core_axis_name="c", subcore_axis_name="s"),
        out_type=[
            pltpu.HBM((NC * NP, D), jnp.float32),
            pltpu.HBM((NC * NP,), jnp.float32),
            pltpu.HBM((NP,), jnp.float32),
        ],
        scratch_types=[
            pltpu.VMEM((IBC, CH), jnp.int32),
            pltpu.VMEM((IBC, CH), jnp.int32),
            pltpu.VMEM((CH, D), jnp.float32),
            pltpu.VMEM((CH, D), jnp.float32),
            pltpu.VMEM((CH,), jnp.float32),
            pltpu.VMEM((ROWS_T,), jnp.float32),
            pltpu.VMEM_SHARED((NP, D), jnp.float32),
            pltpu.VMEM_SHARED((NP,), jnp.float32),
            pltpu.VMEM_SHARED((NP,), jnp.float32),
        ] + [pltpu.SemaphoreType.DMA] * 5,
    )
    return kfn(src2d, dst2d, x_pad)


ROWS_B = 1024
GRID = NP // ROWS_B


def _tc_kernel(x_ref, a0_ref, a1_ref, inv_ref, w0_ref, w1_ref,
               wl1_ref, bl1_ref, wr1_ref, wl2_ref, bl2_ref, wr2_ref,
               out_ref, sh_acc, s2_acc):
    i = pl.program_id(0)

    @pl.when(i == 0)
    def _():
        sh_acc[...] = jnp.zeros((1, D), jnp.float32)
        s2_acc[...] = jnp.zeros((1, D), jnp.float32)
        out_ref[...] = jnp.zeros((1, D_OUT), jnp.float32)

    mm = functools.partial(lax.dot_general,
                           preferred_element_type=jnp.float32,
                           precision=lax.Precision.HIGHEST)
    eye = (lax.broadcasted_iota(jnp.int32, (D, D), 0) ==
           lax.broadcasted_iota(jnp.int32, (D, D), 1)).astype(jnp.float32)
    sh_l = jnp.zeros((1, D), jnp.float32)
    s2_l = jnp.zeros((1, D), jnp.float32)
    for a in range(ROWS_B // D):
        sl = pl.ds(a * D, D)
        agg = a0_ref[0, sl, :] + a1_ref[0, sl, :]
        inv_row = inv_ref[pl.ds(a, 1), :]
        mean = mm(eye * inv_row, agg, (((1,), (0,)), ((), ())))
        hp = mm(mean, wl1_ref[...], (((1,), (1,)), ((), ())))
        hp += mm(x_ref[sl, :], wr1_ref[...], (((1,), (1,)), ((), ())))
        h = jnp.maximum(hp + bl1_ref[...], 0.0)
        row = i * ROWS_B + a * D + lax.broadcasted_iota(jnp.int32, (D, 1), 0)
        h = jnp.where(row < N, h, 0.0)
        w_row = w0_ref[0, pl.ds(a, 1), :] + w1_ref[0, pl.ds(a, 1), :]
        sh_l += jnp.sum(h, axis=0, keepdims=True)
        s2_l += mm(w_row, h, (((1,), (0,)), ((), ())))
    sh_acc[...] += sh_l
    s2_acc[...] += s2_l

    @pl.when(i == GRID - 1)
    def _():
        invn = 1.0 / float(N)
        pooled = lax.dot_general(s2_acc[...] * invn, wl2_ref[...],
                                 (((1,), (1,)), ((), ())),
                                 preferred_element_type=jnp.float32,
                                 precision=lax.Precision.HIGHEST)
        pooled += lax.dot_general(sh_acc[...] * invn, wr2_ref[...],
                                  (((1,), (1,)), ((), ())),
                                  preferred_element_type=jnp.float32,
                                  precision=lax.Precision.HIGHEST)
        pooled += bl2_ref[...]
        m = jnp.max(pooled, axis=-1, keepdims=True)
        e = jnp.exp(pooled - m)
        out_ref[...] = e / jnp.sum(e, axis=-1, keepdims=True)


def _tc_finish(x_pad, agg_parts, w_parts, invcnt,
               W_l1, b_l1, W_r1, W_l2, b_l2, W_r2):
    inv2d = invcnt.reshape(NP // D, D)
    w2d = w_parts.reshape(NC, NP // D, D)
    a3d = agg_parts.reshape(NC, NP, D)
    full = lambda shape: pl.BlockSpec(shape, lambda i: (0,) * len(shape))
    return pl.pallas_call(
        _tc_kernel,
        grid=(GRID,),
        in_specs=[
            pl.BlockSpec((ROWS_B, D), lambda i: (i, 0)),
            pl.BlockSpec((1, ROWS_B, D), lambda i: (0, i, 0)),
            pl.BlockSpec((1, ROWS_B, D), lambda i: (1, i, 0)),
            pl.BlockSpec((ROWS_B // D, D), lambda i: (i, 0)),
            pl.BlockSpec((1, ROWS_B // D, D), lambda i: (0, i, 0)),
            pl.BlockSpec((1, ROWS_B // D, D), lambda i: (1, i, 0)),
            full((D, D)), full((1, D)), full((D, D)),
            full((D_OUT, D)), full((1, D_OUT)), full((D_OUT, D)),
        ],
        out_specs=pl.BlockSpec((1, D_OUT), lambda i: (0, 0)),
        out_shape=jax.ShapeDtypeStruct((1, D_OUT), jnp.float32),
        scratch_shapes=[pltpu.VMEM((1, D), jnp.float32),
                        pltpu.VMEM((1, D), jnp.float32)],
    )(x_pad, a3d, a3d, inv2d, w2d, w2d,
      W_l1, b_l1.reshape(1, D), W_r1, W_l2, b_l2.reshape(1, D_OUT), W_r2)


def kernel(x, edge_index, W_l1, b_l1, W_r1, W_l2, b_l2, W_r2):
    x_pad = jnp.pad(x, ((0, NP - N), (0, 0)))
    spread = N + (jnp.arange(EPAD - E, dtype=jnp.int32) % (NP - N))
    ei = jnp.concatenate([edge_index, jnp.stack([spread, spread])], axis=1)
    src2d = ei[0].reshape(NW, RPT, CH)
    dst2d = ei[1].reshape(NW, RPT, CH)
    agg_parts, w_parts, invcnt = _sc_aggregate(x_pad, src2d, dst2d)
    return _tc_finish(x_pad, agg_parts, w_parts, invcnt,
                      W_l1, b_l1, W_r1, W_l2, b_l2, W_r2)
